# Initial kernel scaffold; baseline (speedup 1.0000x reference)
import jax
import jax.numpy as jnp
from jax.experimental import pallas as pl


def kernel(h, edge_index, W1, W2, mlp_gamma, mlp_beta, out_gamma, out_beta):
    raise NotImplementedError("write your pallas kernel here")


# speedup vs baseline: 3.7484x; 3.7484x over previous
import functools

import jax
import jax.numpy as jnp
from jax import lax
from jax.experimental import pallas as pl
from jax.experimental.pallas import tpu as pltpu
from jax.experimental.pallas import tpu_sc as plsc

N = 10000
E = 320000
D = 128
L = 4

NC = 2
NS = 16
NW = NC * NS
CHUNK = 128
EPW = E // NW
GROUPS = (EPW + CHUNK - 1) // CHUNK
GROUPS = -(-EPW // CHUNK)
EPW_PAD = GROUPS * CHUNK
N_PAD = 10112
ZROWS = N_PAD // NS
WROWS = 624
WTAIL = N - NS * WROWS


def _make_sc_seg_sum():
    mesh = plsc.VectorSubcoreMesh(core_axis_name="c", subcore_axis_name="s")

    def body(h_hbm, src_hbm, dst_hbm, zeros_hbm, out_hbm,
             agg_spmem, src_v, dst_v, rows_v, sem):
        c = lax.axis_index("c")
        s = lax.axis_index("s")
        wid = c * NS + s

        pltpu.sync_copy(zeros_hbm, agg_spmem.at[pl.ds(s * ZROWS, ZROWS)])
        plsc.subcore_barrier()

        pltpu.sync_copy(src_hbm.at[wid], src_v)
        pltpu.sync_copy(dst_hbm.at[wid], dst_v)

        def step(g, carry):
            pltpu.async_copy(h_hbm.at[src_v.at[g]], rows_v, sem).wait()
            pltpu.sync_copy(rows_v, agg_spmem.at[dst_v.at[g]], add=True)
            return carry

        lax.fori_loop(0, GROUPS, step, 0)
        plsc.subcore_barrier()

        pltpu.sync_copy(agg_spmem.at[pl.ds(s * WROWS, WROWS)],
                        out_hbm.at[c, pl.ds(s * WROWS, WROWS)])

        @pl.when(s == NS - 1)
        def _():
            pltpu.sync_copy(agg_spmem.at[pl.ds(NS * WROWS, WTAIL)],
                            out_hbm.at[c, pl.ds(NS * WROWS, WTAIL)])

    return pl.kernel(
        body,
        out_type=jax.ShapeDtypeStruct((NC, N, D), jnp.float32),
        mesh=mesh,
        scratch_types=[
            pltpu.VMEM_SHARED((N_PAD, D), jnp.float32),
            pltpu.VMEM((GROUPS, CHUNK), jnp.int32),
            pltpu.VMEM((GROUPS, CHUNK), jnp.int32),
            pltpu.VMEM((CHUNK, D), jnp.float32),
            pltpu.SemaphoreType.DMA,
        ],
    )


_sc_seg_sum = _make_sc_seg_sum()


ROWS_BLK = 1000
NB = N // ROWS_BLK


def _tc_layer_body(h_ref, agg_ref, w1_ref, w2_ref, g1_ref, b1_ref,
                   g2_ref, b2_ref, out_ref, y_scr, st_scr):
    p = pl.program_id(0)
    i = pl.program_id(1)
    cdims = (((1,), (1,)), ((), ()))

    @pl.when(p == 0)
    def _():
        x = h_ref[...] + agg_ref[0] + agg_ref[1]
        y = lax.dot_general(x, w1_ref[...], cdims,
                            preferred_element_type=jnp.float32)
        y_scr[pl.ds(i * ROWS_BLK, ROWS_BLK), :] = y
        cs = jnp.sum(y, axis=0, keepdims=True)
        cq = jnp.sum(y * y, axis=0, keepdims=True)

        @pl.when(i == 0)
        def _():
            st_scr[0:1, :] = cs
            st_scr[1:2, :] = cq

        @pl.when(i > 0)
        def _():
            st_scr[0:1, :] += cs
            st_scr[1:2, :] += cq

    @pl.when(p == 1)
    def _():
        m = st_scr[0:1, :] / N
        v = st_scr[1:2, :] / N - m * m
        s1 = g1_ref[...] * lax.rsqrt(v + 1e-5)
        t1 = b1_ref[...] - m * s1
        y = y_scr[pl.ds(i * ROWS_BLK, ROWS_BLK), :]
        u = y * s1 + t1
        u = jnp.where(u >= 0, u, 0.01 * u)
        z = lax.dot_general(u, w2_ref[...], cdims,
                            preferred_element_type=jnp.float32)
        y_scr[pl.ds(i * ROWS_BLK, ROWS_BLK), :] = z
        cs = jnp.sum(z, axis=0, keepdims=True)
        cq = jnp.sum(z * z, axis=0, keepdims=True)

        @pl.when(i == 0)
        def _():
            st_scr[2:3, :] = cs
            st_scr[3:4, :] = cq

        @pl.when(i > 0)
        def _():
            st_scr[2:3, :] += cs
            st_scr[3:4, :] += cq

    @pl.when(p == 2)
    def _():
        m = st_scr[2:3, :] / N
        v = st_scr[3:4, :] / N - m * m
        s2 = g2_ref[...] * lax.rsqrt(v + 1e-5)
        t2 = b2_ref[...] - m * s2
        z = y_scr[pl.ds(i * ROWS_BLK, ROWS_BLK), :]
        o = z * s2 + t2
        out_ref[...] = jnp.where(o >= 0, o, 0.01 * o)


def _tc_layer(h, agg2, w1, w2, g1, b1, g2, b2):
    vec = lambda: pl.BlockSpec((1, D), lambda p, i: (0, 0))
    return pl.pallas_call(
        _tc_layer_body,
        grid=(3, NB),
        in_specs=[
            pl.BlockSpec((ROWS_BLK, D), lambda p, i: (i, 0)),
            pl.BlockSpec((NC, ROWS_BLK, D), lambda p, i: (0, i, 0)),
            pl.BlockSpec((D, D), lambda p, i: (0, 0)),
            pl.BlockSpec((D, D), lambda p, i: (0, 0)),
            vec(), vec(), vec(), vec(),
        ],
        out_specs=pl.BlockSpec((ROWS_BLK, D), lambda p, i: (i, 0)),
        out_shape=jax.ShapeDtypeStruct((N, D), jnp.float32),
        scratch_shapes=[
            pltpu.VMEM((N, D), jnp.float32),
            pltpu.VMEM((8, D), jnp.float32),
        ],
    )(h, agg2, w1, w2, g1, b1, g2, b2)


def kernel(h, edge_index, W1, W2, mlp_gamma, mlp_beta, out_gamma, out_beta):
    src = edge_index[0]
    dst = edge_index[1]
    pad = NW * EPW_PAD - E
    src_p = jnp.concatenate([src, jnp.zeros((pad,), jnp.int32)])
    dst_p = jnp.concatenate([dst, jnp.full((pad,), N, jnp.int32)])
    src_r = src_p.reshape(NW, GROUPS, CHUNK)
    dst_r = dst_p.reshape(NW, GROUPS, CHUNK)
    zeros_hbm = jnp.zeros((ZROWS, D), jnp.float32)

    g1 = mlp_gamma.reshape(L, 1, D)
    b1 = mlp_beta.reshape(L, 1, D)
    g2 = out_gamma.reshape(L, 1, D)
    b2 = out_beta.reshape(L, 1, D)

    hs = [h]
    for i in range(L):
        agg2 = _sc_seg_sum(h, src_r, dst_r, zeros_hbm)
        h = _tc_layer(h, agg2, W1[i], W2[i], g1[i], b1[i], g2[i], b2[i])
        hs.append(h)
    return jnp.concatenate(hs, axis=-1)


# speedup vs baseline: 4.3269x; 1.1543x over previous
import functools

import jax
import jax.numpy as jnp
from jax import lax
from jax.experimental import pallas as pl
from jax.experimental.pallas import tpu as pltpu
from jax.experimental.pallas import tpu_sc as plsc

N = 10000
E = 320000
D = 128
L = 4

NC = 2
NS = 16
NW = NC * NS
CHUNK = 128
NBUF = 2
EPW = E // NW
GROUPS = -(-EPW // CHUNK)
EPW_PAD = GROUPS * CHUNK
N_PAD = 10112
ZROWS = N_PAD // NS
WROWS = 624
WTAIL = N - NS * WROWS


def _make_sc_seg_sum():
    mesh = plsc.VectorSubcoreMesh(core_axis_name="c", subcore_axis_name="s")

    def body(h_hbm, src_hbm, dst_hbm, zeros_hbm, out_hbm,
             agg_spmem, src_i, dst_i, rows_v, isem, gsem):
        c = lax.axis_index("c")
        s = lax.axis_index("s")
        wid = c * NS + s

        pltpu.sync_copy(zeros_hbm, agg_spmem.at[pl.ds(s * ZROWS, ZROWS)])
        plsc.subcore_barrier()

        def stage_idx(g, slot):
            pltpu.async_copy(src_hbm.at[wid, g], src_i.at[slot], isem.at[slot])
            pltpu.async_copy(dst_hbm.at[wid, g], dst_i.at[slot], isem.at[slot])

        def wait_idx(g, slot):
            pltpu.make_async_copy(src_hbm.at[wid, g], src_i.at[slot],
                                  isem.at[slot]).wait()
            pltpu.make_async_copy(dst_hbm.at[wid, g], dst_i.at[slot],
                                  isem.at[slot]).wait()

        for g in range(NBUF):
            stage_idx(g, g)
        wait_idx(0, 0)
        pltpu.async_copy(h_hbm.at[src_i.at[0]], rows_v.at[0], gsem.at[0])

        def step(g, carry):
            b = lax.rem(g, NBUF)
            nb = lax.rem(g + 1, NBUF)

            @pl.when(g + 1 < GROUPS)
            def _():
                wait_idx(g + 1, nb)
                pltpu.async_copy(h_hbm.at[src_i.at[nb]], rows_v.at[nb],
                                 gsem.at[nb])

            pltpu.make_async_copy(h_hbm.at[src_i.at[b]], rows_v.at[b],
                                  gsem.at[b]).wait()
            pltpu.sync_copy(rows_v.at[b], agg_spmem.at[dst_i.at[b]], add=True)

            @pl.when(g + NBUF < GROUPS)
            def _():
                stage_idx(g + NBUF, b)

            return carry

        lax.fori_loop(0, GROUPS, step, 0)
        plsc.subcore_barrier()

        pltpu.sync_copy(agg_spmem.at[pl.ds(s * WROWS, WROWS)],
                        out_hbm.at[c, pl.ds(s * WROWS, WROWS)])

        @pl.when(s == NS - 1)
        def _():
            pltpu.sync_copy(agg_spmem.at[pl.ds(NS * WROWS, WTAIL)],
                            out_hbm.at[c, pl.ds(NS * WROWS, WTAIL)])

    return pl.kernel(
        body,
        out_type=jax.ShapeDtypeStruct((NC, N, D), jnp.float32),
        mesh=mesh,
        scratch_types=[
            pltpu.VMEM_SHARED((N_PAD, D), jnp.float32),
            pltpu.VMEM((NBUF, CHUNK), jnp.int32),
            pltpu.VMEM((NBUF, CHUNK), jnp.int32),
            pltpu.VMEM((NBUF, CHUNK, D), jnp.float32),
            pltpu.SemaphoreType.DMA((NBUF,)),
            pltpu.SemaphoreType.DMA((NBUF,)),
        ],
    )


_sc_seg_sum = _make_sc_seg_sum()


ROWS_BLK = 1000
NB = N // ROWS_BLK


def _tc_layer_body(h_ref, agg_ref, w1_ref, w2_ref, g1_ref, b1_ref,
                   g2_ref, b2_ref, out_ref, y_scr, st_scr):
    p = pl.program_id(0)
    i = pl.program_id(1)
    cdims = (((1,), (1,)), ((), ()))

    @pl.when(p == 0)
    def _():
        x = h_ref[...] + agg_ref[0] + agg_ref[1]
        y = lax.dot_general(x, w1_ref[...], cdims,
                            preferred_element_type=jnp.float32)
        y_scr[pl.ds(i * ROWS_BLK, ROWS_BLK), :] = y
        cs = jnp.sum(y, axis=0, keepdims=True)
        cq = jnp.sum(y * y, axis=0, keepdims=True)

        @pl.when(i == 0)
        def _():
            st_scr[0:1, :] = cs
            st_scr[1:2, :] = cq

        @pl.when(i > 0)
        def _():
            st_scr[0:1, :] += cs
            st_scr[1:2, :] += cq

    @pl.when(p == 1)
    def _():
        m = st_scr[0:1, :] / N
        v = st_scr[1:2, :] / N - m * m
        s1 = g1_ref[...] * lax.rsqrt(v + 1e-5)
        t1 = b1_ref[...] - m * s1
        y = y_scr[pl.ds(i * ROWS_BLK, ROWS_BLK), :]
        u = y * s1 + t1
        u = jnp.where(u >= 0, u, 0.01 * u)
        z = lax.dot_general(u, w2_ref[...], cdims,
                            preferred_element_type=jnp.float32)
        y_scr[pl.ds(i * ROWS_BLK, ROWS_BLK), :] = z
        cs = jnp.sum(z, axis=0, keepdims=True)
        cq = jnp.sum(z * z, axis=0, keepdims=True)

        @pl.when(i == 0)
        def _():
            st_scr[2:3, :] = cs
            st_scr[3:4, :] = cq

        @pl.when(i > 0)
        def _():
            st_scr[2:3, :] += cs
            st_scr[3:4, :] += cq

    @pl.when(p == 2)
    def _():
        m = st_scr[2:3, :] / N
        v = st_scr[3:4, :] / N - m * m
        s2 = g2_ref[...] * lax.rsqrt(v + 1e-5)
        t2 = b2_ref[...] - m * s2
        z = y_scr[pl.ds(i * ROWS_BLK, ROWS_BLK), :]
        o = z * s2 + t2
        out_ref[...] = jnp.where(o >= 0, o, 0.01 * o)


def _tc_layer(h, agg2, w1, w2, g1, b1, g2, b2):
    vec = lambda: pl.BlockSpec((1, D), lambda p, i: (0, 0))
    return pl.pallas_call(
        _tc_layer_body,
        grid=(3, NB),
        in_specs=[
            pl.BlockSpec((ROWS_BLK, D), lambda p, i: (i, 0)),
            pl.BlockSpec((NC, ROWS_BLK, D), lambda p, i: (0, i, 0)),
            pl.BlockSpec((D, D), lambda p, i: (0, 0)),
            pl.BlockSpec((D, D), lambda p, i: (0, 0)),
            vec(), vec(), vec(), vec(),
        ],
        out_specs=pl.BlockSpec((ROWS_BLK, D), lambda p, i: (i, 0)),
        out_shape=jax.ShapeDtypeStruct((N, D), jnp.float32),
        scratch_shapes=[
            pltpu.VMEM((N, D), jnp.float32),
            pltpu.VMEM((8, D), jnp.float32),
        ],
    )(h, agg2, w1, w2, g1, b1, g2, b2)


def kernel(h, edge_index, W1, W2, mlp_gamma, mlp_beta, out_gamma, out_beta):
    src = edge_index[0]
    dst = edge_index[1]
    pad = NW * EPW_PAD - E
    src_p = jnp.concatenate([src, jnp.zeros((pad,), jnp.int32)])
    dst_p = jnp.concatenate([dst, jnp.full((pad,), N, jnp.int32)])
    src_r = src_p.reshape(NW, GROUPS, CHUNK)
    dst_r = dst_p.reshape(NW, GROUPS, CHUNK)
    zeros_hbm = jnp.zeros((ZROWS, D), jnp.float32)

    g1 = mlp_gamma.reshape(L, 1, D)
    b1 = mlp_beta.reshape(L, 1, D)
    g2 = out_gamma.reshape(L, 1, D)
    b2 = out_beta.reshape(L, 1, D)

    hs = [h]
    for i in range(L):
        agg2 = _sc_seg_sum(h, src_r, dst_r, zeros_hbm)
        h = _tc_layer(h, agg2, W1[i], W2[i], g1[i], b1[i], g2[i], b2[i])
        hs.append(h)
    return jnp.concatenate(hs, axis=-1)


# speedup vs baseline: 4.4809x; 1.0356x over previous
import functools

import jax
import jax.numpy as jnp
from jax import lax
from jax.experimental import pallas as pl
from jax.experimental.pallas import tpu as pltpu
from jax.experimental.pallas import tpu_sc as plsc

N = 10000
E = 320000
D = 128
L = 4

NC = 1
NS = 16
NW = NC * NS
CHUNK = 128
NBUF = 2
EPW = E // NW
GROUPS = -(-EPW // CHUNK)
EPW_PAD = GROUPS * CHUNK
N_PAD = 10112
ZROWS = N_PAD // NS
WROWS = 624
WTAIL = N - NS * WROWS


def _make_sc_seg_sum():
    mesh = plsc.VectorSubcoreMesh(core_axis_name="c", subcore_axis_name="s",
                                  num_cores=NC)

    def body(h_hbm, src_hbm, dst_hbm, zeros_hbm, out_hbm,
             agg_spmem, src_i, dst_i, rows_v, isem, gsem):
        c = lax.axis_index("c")
        s = lax.axis_index("s")
        wid = c * NS + s

        pltpu.sync_copy(zeros_hbm, agg_spmem.at[pl.ds(s * ZROWS, ZROWS)])
        plsc.subcore_barrier()

        def stage_idx(g, slot):
            pltpu.async_copy(src_hbm.at[wid, g], src_i.at[slot], isem.at[slot])
            pltpu.async_copy(dst_hbm.at[wid, g], dst_i.at[slot], isem.at[slot])

        def wait_idx(g, slot):
            pltpu.make_async_copy(src_hbm.at[wid, g], src_i.at[slot],
                                  isem.at[slot]).wait()
            pltpu.make_async_copy(dst_hbm.at[wid, g], dst_i.at[slot],
                                  isem.at[slot]).wait()

        for g in range(NBUF):
            stage_idx(g, g)
        wait_idx(0, 0)
        pltpu.async_copy(h_hbm.at[src_i.at[0]], rows_v.at[0], gsem.at[0])

        def step(g, carry):
            b = lax.rem(g, NBUF)
            nb = lax.rem(g + 1, NBUF)

            @pl.when(g + 1 < GROUPS)
            def _():
                wait_idx(g + 1, nb)
                pltpu.async_copy(h_hbm.at[src_i.at[nb]], rows_v.at[nb],
                                 gsem.at[nb])

            pltpu.make_async_copy(h_hbm.at[src_i.at[b]], rows_v.at[b],
                                  gsem.at[b]).wait()
            pltpu.sync_copy(rows_v.at[b], agg_spmem.at[dst_i.at[b]], add=True)

            @pl.when(g + NBUF < GROUPS)
            def _():
                stage_idx(g + NBUF, b)

            return carry

        lax.fori_loop(0, GROUPS, step, 0)
        plsc.subcore_barrier()

        pltpu.sync_copy(agg_spmem.at[pl.ds(s * WROWS, WROWS)],
                        out_hbm.at[c, pl.ds(s * WROWS, WROWS)])

        @pl.when(s == NS - 1)
        def _():
            pltpu.sync_copy(agg_spmem.at[pl.ds(NS * WROWS, WTAIL)],
                            out_hbm.at[c, pl.ds(NS * WROWS, WTAIL)])

    return pl.kernel(
        body,
        out_type=jax.ShapeDtypeStruct((NC, N, D), jnp.float32),
        mesh=mesh,
        scratch_types=[
            pltpu.VMEM_SHARED((N_PAD, D), jnp.float32),
            pltpu.VMEM((NBUF, CHUNK), jnp.int32),
            pltpu.VMEM((NBUF, CHUNK), jnp.int32),
            pltpu.VMEM((NBUF, CHUNK, D), jnp.float32),
            pltpu.SemaphoreType.DMA((NBUF,)),
            pltpu.SemaphoreType.DMA((NBUF,)),
        ],
    )


_sc_seg_sum = _make_sc_seg_sum()


ROWS_BLK = 1000
NB = N // ROWS_BLK


def _tc_layer_body(h_ref, agg_ref, w1_ref, w2_ref, g1_ref, b1_ref,
                   g2_ref, b2_ref, out_ref, y_scr, st_scr):
    p = pl.program_id(0)
    i = pl.program_id(1)
    cdims = (((1,), (1,)), ((), ()))

    @pl.when(p == 0)
    def _():
        x = h_ref[...] + jnp.sum(agg_ref[...], axis=0)
        y = lax.dot_general(x, w1_ref[...], cdims,
                            preferred_element_type=jnp.float32)
        y_scr[pl.ds(i * ROWS_BLK, ROWS_BLK), :] = y
        cs = jnp.sum(y, axis=0, keepdims=True)
        cq = jnp.sum(y * y, axis=0, keepdims=True)

        @pl.when(i == 0)
        def _():
            st_scr[0:1, :] = cs
            st_scr[1:2, :] = cq

        @pl.when(i > 0)
        def _():
            st_scr[0:1, :] += cs
            st_scr[1:2, :] += cq

    @pl.when(p == 1)
    def _():
        m = st_scr[0:1, :] / N
        v = st_scr[1:2, :] / N - m * m
        s1 = g1_ref[...] * lax.rsqrt(v + 1e-5)
        t1 = b1_ref[...] - m * s1
        y = y_scr[pl.ds(i * ROWS_BLK, ROWS_BLK), :]
        u = y * s1 + t1
        u = jnp.where(u >= 0, u, 0.01 * u)
        z = lax.dot_general(u, w2_ref[...], cdims,
                            preferred_element_type=jnp.float32)
        y_scr[pl.ds(i * ROWS_BLK, ROWS_BLK), :] = z
        cs = jnp.sum(z, axis=0, keepdims=True)
        cq = jnp.sum(z * z, axis=0, keepdims=True)

        @pl.when(i == 0)
        def _():
            st_scr[2:3, :] = cs
            st_scr[3:4, :] = cq

        @pl.when(i > 0)
        def _():
            st_scr[2:3, :] += cs
            st_scr[3:4, :] += cq

    @pl.when(p == 2)
    def _():
        m = st_scr[2:3, :] / N
        v = st_scr[3:4, :] / N - m * m
        s2 = g2_ref[...] * lax.rsqrt(v + 1e-5)
        t2 = b2_ref[...] - m * s2
        z = y_scr[pl.ds(i * ROWS_BLK, ROWS_BLK), :]
        o = z * s2 + t2
        out_ref[...] = jnp.where(o >= 0, o, 0.01 * o)


def _tc_layer(h, agg2, w1, w2, g1, b1, g2, b2):
    vec = lambda: pl.BlockSpec((1, D), lambda p, i: (0, 0))
    return pl.pallas_call(
        _tc_layer_body,
        grid=(3, NB),
        in_specs=[
            pl.BlockSpec((ROWS_BLK, D), lambda p, i: (i, 0)),
            pl.BlockSpec((NC, ROWS_BLK, D), lambda p, i: (0, i, 0)),
            pl.BlockSpec((D, D), lambda p, i: (0, 0)),
            pl.BlockSpec((D, D), lambda p, i: (0, 0)),
            vec(), vec(), vec(), vec(),
        ],
        out_specs=pl.BlockSpec((ROWS_BLK, D), lambda p, i: (i, 0)),
        out_shape=jax.ShapeDtypeStruct((N, D), jnp.float32),
        scratch_shapes=[
            pltpu.VMEM((N, D), jnp.float32),
            pltpu.VMEM((8, D), jnp.float32),
        ],
    )(h, agg2, w1, w2, g1, b1, g2, b2)


def kernel(h, edge_index, W1, W2, mlp_gamma, mlp_beta, out_gamma, out_beta):
    src = edge_index[0]
    dst = edge_index[1]
    pad = NW * EPW_PAD - E
    src_p = jnp.concatenate([src, jnp.zeros((pad,), jnp.int32)])
    dst_p = jnp.concatenate([dst, jnp.full((pad,), N, jnp.int32)])
    src_r = src_p.reshape(NW, GROUPS, CHUNK)
    dst_r = dst_p.reshape(NW, GROUPS, CHUNK)
    zeros_hbm = jnp.zeros((ZROWS, D), jnp.float32)

    g1 = mlp_gamma.reshape(L, 1, D)
    b1 = mlp_beta.reshape(L, 1, D)
    g2 = out_gamma.reshape(L, 1, D)
    b2 = out_beta.reshape(L, 1, D)

    hs = [h]
    for i in range(L):
        agg2 = _sc_seg_sum(h, src_r, dst_r, zeros_hbm)
        h = _tc_layer(h, agg2, W1[i], W2[i], g1[i], b1[i], g2[i], b2[i])
        hs.append(h)
    return jnp.concatenate(hs, axis=-1)


# speedup vs baseline: 4.6041x; 1.0275x over previous
import functools

import jax
import jax.numpy as jnp
from jax import lax
from jax.experimental import pallas as pl
from jax.experimental.pallas import tpu as pltpu
from jax.experimental.pallas import tpu_sc as plsc

N = 10000
E = 320000
D = 128
L = 4

NC = 1
NS = 16
NW = NC * NS
CHUNK = 128
NBUF = 2
EPW = E // NW
GROUPS = -(-EPW // CHUNK)
EPW_PAD = GROUPS * CHUNK
N_PAD = 10112
ZROWS = N_PAD // NS
WROWS = 624
WTAIL = N - NS * WROWS


def _make_sc_seg_sum():
    mesh = plsc.VectorSubcoreMesh(core_axis_name="c", subcore_axis_name="s",
                                  num_cores=NC)

    def body(h_hbm, src_hbm, dst_hbm, zeros_hbm, out_hbm,
             agg_spmem, src_i, dst_i, rows_v, isem, gsem):
        c = lax.axis_index("c")
        s = lax.axis_index("s")
        wid = c * NS + s

        pltpu.sync_copy(zeros_hbm, agg_spmem.at[pl.ds(s * ZROWS, ZROWS)])
        plsc.subcore_barrier()

        def stage_idx(g, slot):
            pltpu.async_copy(src_hbm.at[wid, g], src_i.at[slot], isem.at[slot])
            pltpu.async_copy(dst_hbm.at[wid, g], dst_i.at[slot], isem.at[slot])

        def wait_idx(g, slot):
            pltpu.make_async_copy(src_hbm.at[wid, g], src_i.at[slot],
                                  isem.at[slot]).wait()
            pltpu.make_async_copy(dst_hbm.at[wid, g], dst_i.at[slot],
                                  isem.at[slot]).wait()

        for g in range(NBUF):
            stage_idx(g, g)
        wait_idx(0, 0)
        pltpu.async_copy(h_hbm.at[src_i.at[0]], rows_v.at[0], gsem.at[0])

        def step(g, carry):
            b = lax.rem(g, NBUF)
            nb = lax.rem(g + 1, NBUF)

            @pl.when(g + 1 < GROUPS)
            def _():
                wait_idx(g + 1, nb)
                pltpu.async_copy(h_hbm.at[src_i.at[nb]], rows_v.at[nb],
                                 gsem.at[nb])

            pltpu.make_async_copy(h_hbm.at[src_i.at[b]], rows_v.at[b],
                                  gsem.at[b]).wait()
            pltpu.sync_copy(rows_v.at[b], agg_spmem.at[dst_i.at[b]], add=True)

            @pl.when(g + NBUF < GROUPS)
            def _():
                stage_idx(g + NBUF, b)

            return carry

        lax.fori_loop(0, GROUPS, step, 0)
        plsc.subcore_barrier()

        pltpu.sync_copy(agg_spmem.at[pl.ds(s * WROWS, WROWS)],
                        out_hbm.at[c, pl.ds(s * WROWS, WROWS)])

        @pl.when(s == NS - 1)
        def _():
            pltpu.sync_copy(agg_spmem.at[pl.ds(NS * WROWS, WTAIL)],
                            out_hbm.at[c, pl.ds(NS * WROWS, WTAIL)])

    return pl.kernel(
        body,
        out_type=jax.ShapeDtypeStruct((NC, N, D), jnp.float32),
        mesh=mesh,
        scratch_types=[
            pltpu.VMEM_SHARED((N_PAD, D), jnp.float32),
            pltpu.VMEM((NBUF, CHUNK), jnp.int32),
            pltpu.VMEM((NBUF, CHUNK), jnp.int32),
            pltpu.VMEM((NBUF, CHUNK, D), jnp.float32),
            pltpu.SemaphoreType.DMA((NBUF,)),
            pltpu.SemaphoreType.DMA((NBUF,)),
        ],
    )


_sc_seg_sum = _make_sc_seg_sum()


ROWS_BLK = 1000
NB = N // ROWS_BLK


def _tc_layer_body(h_ref, agg_ref, w1_ref, w2_ref, g1_ref, b1_ref,
                   g2_ref, b2_ref, out_ref, y_scr, st_scr):
    p = pl.program_id(0)
    i = pl.program_id(1)
    cdims = (((1,), (1,)), ((), ()))

    @pl.when(p == 0)
    def _():
        x = h_ref[...] + jnp.sum(agg_ref[...], axis=0)
        y = lax.dot_general(x, w1_ref[...], cdims,
                            preferred_element_type=jnp.float32)
        y_scr[pl.ds(i * ROWS_BLK, ROWS_BLK), :] = y
        cs = jnp.sum(y, axis=0, keepdims=True)
        cq = jnp.sum(y * y, axis=0, keepdims=True)

        @pl.when(i == 0)
        def _():
            st_scr[0:1, :] = cs
            st_scr[1:2, :] = cq

        @pl.when(i > 0)
        def _():
            st_scr[0:1, :] += cs
            st_scr[1:2, :] += cq

    @pl.when(p == 1)
    def _():
        m = st_scr[0:1, :] / N
        v = st_scr[1:2, :] / N - m * m
        s1 = g1_ref[...] * lax.rsqrt(v + 1e-5)
        t1 = b1_ref[...] - m * s1
        y = y_scr[pl.ds(i * ROWS_BLK, ROWS_BLK), :]
        u = y * s1 + t1
        u = jnp.where(u >= 0, u, 0.01 * u)
        z = lax.dot_general(u, w2_ref[...], cdims,
                            preferred_element_type=jnp.float32)
        y_scr[pl.ds(i * ROWS_BLK, ROWS_BLK), :] = z
        cs = jnp.sum(z, axis=0, keepdims=True)
        cq = jnp.sum(z * z, axis=0, keepdims=True)

        @pl.when(i == 0)
        def _():
            st_scr[2:3, :] = cs
            st_scr[3:4, :] = cq

        @pl.when(i > 0)
        def _():
            st_scr[2:3, :] += cs
            st_scr[3:4, :] += cq

    @pl.when(p == 2)
    def _():
        m = st_scr[2:3, :] / N
        v = st_scr[3:4, :] / N - m * m
        s2 = g2_ref[...] * lax.rsqrt(v + 1e-5)
        t2 = b2_ref[...] - m * s2
        z = y_scr[pl.ds(i * ROWS_BLK, ROWS_BLK), :]
        o = z * s2 + t2
        out_ref[...] = jnp.where(o >= 0, o, 0.01 * o)


def _tc_layer(h, agg2, w1, w2, g1, b1, g2, b2):
    vec = lambda: pl.BlockSpec((1, D), lambda p, i: (0, 0))
    return pl.pallas_call(
        _tc_layer_body,
        grid=(3, NB),
        in_specs=[
            pl.BlockSpec((ROWS_BLK, D),
                         lambda p, i: (jnp.where(p == 0, i, 0), 0)),
            pl.BlockSpec((NC, ROWS_BLK, D),
                         lambda p, i: (0, jnp.where(p == 0, i, 0), 0)),
            pl.BlockSpec((D, D), lambda p, i: (0, 0)),
            pl.BlockSpec((D, D), lambda p, i: (0, 0)),
            vec(), vec(), vec(), vec(),
        ],
        out_specs=pl.BlockSpec((ROWS_BLK, D),
                               lambda p, i: (jnp.where(p == 2, i, 0), 0)),
        out_shape=jax.ShapeDtypeStruct((N, D), jnp.float32),
        scratch_shapes=[
            pltpu.VMEM((N, D), jnp.float32),
            pltpu.VMEM((8, D), jnp.float32),
        ],
    )(h, agg2, w1, w2, g1, b1, g2, b2)


def kernel(h, edge_index, W1, W2, mlp_gamma, mlp_beta, out_gamma, out_beta):
    src = edge_index[0]
    dst = edge_index[1]
    pad = NW * EPW_PAD - E
    src_p = jnp.concatenate([src, jnp.zeros((pad,), jnp.int32)])
    dst_p = jnp.concatenate([dst, jnp.full((pad,), N, jnp.int32)])
    src_r = src_p.reshape(NW, GROUPS, CHUNK)
    dst_r = dst_p.reshape(NW, GROUPS, CHUNK)
    zeros_hbm = jnp.zeros((ZROWS, D), jnp.float32)

    g1 = mlp_gamma.reshape(L, 1, D)
    b1 = mlp_beta.reshape(L, 1, D)
    g2 = out_gamma.reshape(L, 1, D)
    b2 = out_beta.reshape(L, 1, D)

    hs = [h]
    for i in range(L):
        agg2 = _sc_seg_sum(h, src_r, dst_r, zeros_hbm)
        h = _tc_layer(h, agg2, W1[i], W2[i], g1[i], b1[i], g2[i], b2[i])
        hs.append(h)
    return jnp.concatenate(hs, axis=-1)


# speedup vs baseline: 5.9868x; 1.3003x over previous
import jax
import jax.numpy as jnp
from jax import lax
from jax.experimental import pallas as pl
from jax.experimental.pallas import tpu as pltpu
from jax.experimental.pallas import tpu_sc as plsc

N = 10000
E = 320000
D = 128
HD = D // 2
L = 4

NC = 2
NS = 16
CHUNK = 128
NBUF = 2
EPT = E // NS
GROUPS = -(-EPT // CHUNK)
EPT_PAD = GROUPS * CHUNK
N_PAD = 10112
ZROWS = N_PAD // NS
BROWS = 624
BTAIL = N - NS * BROWS


def _make_sc_seg_sum():
    mesh = plsc.VectorSubcoreMesh(core_axis_name="c", subcore_axis_name="s",
                                  num_cores=NC)

    def body(hh_hbm, src_hbm, dst_hbm, zeros_hbm, out_hbm,
             h_sp, agg_sp, src_i, dst_i, rows_v, isem, gsem):
        c = lax.axis_index("c")
        s = lax.axis_index("s")

        pltpu.sync_copy(zeros_hbm, agg_sp.at[pl.ds(s * ZROWS, ZROWS)])
        pltpu.sync_copy(hh_hbm.at[c, pl.ds(s * BROWS, BROWS)],
                        h_sp.at[pl.ds(s * BROWS, BROWS)])

        @pl.when(s == NS - 1)
        def _():
            pltpu.sync_copy(hh_hbm.at[c, pl.ds(NS * BROWS, BTAIL)],
                            h_sp.at[pl.ds(NS * BROWS, BTAIL)])

        plsc.subcore_barrier()

        def stage_idx(g, slot):
            pltpu.async_copy(src_hbm.at[s, g], src_i.at[slot], isem.at[slot])
            pltpu.async_copy(dst_hbm.at[s, g], dst_i.at[slot], isem.at[slot])

        def wait_idx(g, slot):
            pltpu.make_async_copy(src_hbm.at[s, g], src_i.at[slot],
                                  isem.at[slot]).wait()
            pltpu.make_async_copy(dst_hbm.at[s, g], dst_i.at[slot],
                                  isem.at[slot]).wait()

        for g in range(NBUF):
            stage_idx(g, g)
        wait_idx(0, 0)
        pltpu.async_copy(h_sp.at[src_i.at[0]], rows_v.at[0], gsem.at[0])

        def step(g, carry):
            b = lax.rem(g, NBUF)
            nb = lax.rem(g + 1, NBUF)

            @pl.when(g + 1 < GROUPS)
            def _():
                wait_idx(g + 1, nb)
                pltpu.async_copy(h_sp.at[src_i.at[nb]], rows_v.at[nb],
                                 gsem.at[nb])

            pltpu.make_async_copy(h_sp.at[src_i.at[b]], rows_v.at[b],
                                  gsem.at[b]).wait()
            pltpu.sync_copy(rows_v.at[b], agg_sp.at[dst_i.at[b]], add=True)

            @pl.when(g + NBUF < GROUPS)
            def _():
                stage_idx(g + NBUF, b)

            return carry

        lax.fori_loop(0, GROUPS, step, 0)
        plsc.subcore_barrier()

        pltpu.sync_copy(agg_sp.at[pl.ds(s * BROWS, BROWS)],
                        out_hbm.at[c, pl.ds(s * BROWS, BROWS)])

        @pl.when(s == NS - 1)
        def _():
            pltpu.sync_copy(agg_sp.at[pl.ds(NS * BROWS, BTAIL)],
                            out_hbm.at[c, pl.ds(NS * BROWS, BTAIL)])

    return pl.kernel(
        body,
        out_type=jax.ShapeDtypeStruct((NC, N, HD), jnp.float32),
        mesh=mesh,
        scratch_types=[
            pltpu.VMEM_SHARED((N, HD), jnp.float32),
            pltpu.VMEM_SHARED((N_PAD, HD), jnp.float32),
            pltpu.VMEM((NBUF, CHUNK), jnp.int32),
            pltpu.VMEM((NBUF, CHUNK), jnp.int32),
            pltpu.VMEM((NBUF, CHUNK, HD), jnp.float32),
            pltpu.SemaphoreType.DMA((NBUF,)),
            pltpu.SemaphoreType.DMA((NBUF,)),
        ],
    )


_sc_seg_sum = _make_sc_seg_sum()


ROWS_BLK = 1000
NB = N // ROWS_BLK


def _tc_layer_body(h_ref, agg_ref, w1_ref, w2_ref, g1_ref, b1_ref,
                   g2_ref, b2_ref, out_ref, hsplit_ref, y_scr, st_scr):
    p = pl.program_id(0)
    i = pl.program_id(1)
    cdims = (((1,), (1,)), ((), ()))

    @pl.when(p == 0)
    def _():
        agg = jnp.concatenate([agg_ref[0], agg_ref[1]], axis=-1)
        x = h_ref[...] + agg
        y = lax.dot_general(x, w1_ref[...], cdims,
                            preferred_element_type=jnp.float32)
        y_scr[pl.ds(i * ROWS_BLK, ROWS_BLK), :] = y
        cs = jnp.sum(y, axis=0, keepdims=True)
        cq = jnp.sum(y * y, axis=0, keepdims=True)

        @pl.when(i == 0)
        def _():
            st_scr[0:1, :] = cs
            st_scr[1:2, :] = cq

        @pl.when(i > 0)
        def _():
            st_scr[0:1, :] += cs
            st_scr[1:2, :] += cq

    @pl.when(p == 1)
    def _():
        m = st_scr[0:1, :] / N
        v = st_scr[1:2, :] / N - m * m
        s1 = g1_ref[...] * lax.rsqrt(v + 1e-5)
        t1 = b1_ref[...] - m * s1
        y = y_scr[pl.ds(i * ROWS_BLK, ROWS_BLK), :]
        u = y * s1 + t1
        u = jnp.where(u >= 0, u, 0.01 * u)
        z = lax.dot_general(u, w2_ref[...], cdims,
                            preferred_element_type=jnp.float32)
        y_scr[pl.ds(i * ROWS_BLK, ROWS_BLK), :] = z
        cs = jnp.sum(z, axis=0, keepdims=True)
        cq = jnp.sum(z * z, axis=0, keepdims=True)

        @pl.when(i == 0)
        def _():
            st_scr[2:3, :] = cs
            st_scr[3:4, :] = cq

        @pl.when(i > 0)
        def _():
            st_scr[2:3, :] += cs
            st_scr[3:4, :] += cq

    @pl.when(p == 2)
    def _():
        m = st_scr[2:3, :] / N
        v = st_scr[3:4, :] / N - m * m
        s2 = g2_ref[...] * lax.rsqrt(v + 1e-5)
        t2 = b2_ref[...] - m * s2
        z = y_scr[pl.ds(i * ROWS_BLK, ROWS_BLK), :]
        o = z * s2 + t2
        o = jnp.where(o >= 0, o, 0.01 * o)
        out_ref[...] = o
        hsplit_ref[0] = o[:, :HD]
        hsplit_ref[1] = o[:, HD:]


def _tc_layer(h, agg2, w1, w2, g1, b1, g2, b2):
    vec = lambda: pl.BlockSpec((1, D), lambda p, i: (0, 0))
    return pl.pallas_call(
        _tc_layer_body,
        grid=(3, NB),
        in_specs=[
            pl.BlockSpec((ROWS_BLK, D),
                         lambda p, i: (jnp.where(p == 0, i, 0), 0)),
            pl.BlockSpec((NC, ROWS_BLK, HD),
                         lambda p, i: (0, jnp.where(p == 0, i, 0), 0)),
            pl.BlockSpec((D, D), lambda p, i: (0, 0)),
            pl.BlockSpec((D, D), lambda p, i: (0, 0)),
            vec(), vec(), vec(), vec(),
        ],
        out_specs=[
            pl.BlockSpec((ROWS_BLK, D),
                         lambda p, i: (jnp.where(p == 2, i, 0), 0)),
            pl.BlockSpec((NC, ROWS_BLK, HD),
                         lambda p, i: (0, jnp.where(p == 2, i, 0), 0)),
        ],
        out_shape=[
            jax.ShapeDtypeStruct((N, D), jnp.float32),
            jax.ShapeDtypeStruct((NC, N, HD), jnp.float32),
        ],
        scratch_shapes=[
            pltpu.VMEM((N, D), jnp.float32),
            pltpu.VMEM((8, D), jnp.float32),
        ],
    )(h, agg2, w1, w2, g1, b1, g2, b2)


def kernel(h, edge_index, W1, W2, mlp_gamma, mlp_beta, out_gamma, out_beta):
    src = edge_index[0]
    dst = edge_index[1]
    pad = NS * EPT_PAD - E
    src_p = jnp.concatenate([src, jnp.zeros((pad,), jnp.int32)])
    dst_p = jnp.concatenate([dst, jnp.full((pad,), N, jnp.int32)])
    src_r = src_p.reshape(NS, GROUPS, CHUNK)
    dst_r = dst_p.reshape(NS, GROUPS, CHUNK)
    zeros_hbm = jnp.zeros((ZROWS, HD), jnp.float32)

    g1 = mlp_gamma.reshape(L, 1, D)
    b1 = mlp_beta.reshape(L, 1, D)
    g2 = out_gamma.reshape(L, 1, D)
    b2 = out_beta.reshape(L, 1, D)

    hh = jnp.stack([h[:, :HD], h[:, HD:]])
    hs = [h]
    for i in range(L):
        agg2 = _sc_seg_sum(hh, src_r, dst_r, zeros_hbm)
        h, hh = _tc_layer(h, agg2, W1[i], W2[i], g1[i], b1[i], g2[i], b2[i])
        hs.append(h)
    return jnp.concatenate(hs, axis=-1)


# speedup vs baseline: 7.8245x; 1.3069x over previous
import jax
import jax.numpy as jnp
from jax import lax
from jax.experimental import pallas as pl
from jax.experimental.pallas import tpu as pltpu
from jax.experimental.pallas import tpu_sc as plsc

N = 10000
E = 320000
D = 128
HD = D // 2
L = 4

NC = 2
NS = 16
CHUNK = 128
NBUF = 2
DBUF = 3
EPT = E // NS
GROUPS = -(-EPT // CHUNK)
EPT_PAD = GROUPS * CHUNK
N_PAD = 10112
ZROWS = N_PAD // NS
BROWS = 624
BTAIL = N - NS * BROWS


def _make_sc_seg_sum():
    mesh = plsc.VectorSubcoreMesh(core_axis_name="c", subcore_axis_name="s",
                                  num_cores=NC)

    def body(hh_hbm, src_hbm, dst_hbm, zeros_hbm, out_hbm,
             h_sp, agg_sp, src_i, dst_i, rows_v, isem, dsem, gsem, ssem):
        c = lax.axis_index("c")
        s = lax.axis_index("s")

        pltpu.sync_copy(zeros_hbm, agg_sp.at[pl.ds(s * ZROWS, ZROWS)])
        pltpu.sync_copy(hh_hbm.at[c, pl.ds(s * BROWS, BROWS)],
                        h_sp.at[pl.ds(s * BROWS, BROWS)])

        @pl.when(s == NS - 1)
        def _():
            pltpu.sync_copy(hh_hbm.at[c, pl.ds(NS * BROWS, BTAIL)],
                            h_sp.at[pl.ds(NS * BROWS, BTAIL)])

        plsc.subcore_barrier()

        def stage_src(g, slot):
            pltpu.async_copy(src_hbm.at[s, g], src_i.at[slot], isem.at[slot])

        def stage_dst(g, slot):
            pltpu.async_copy(dst_hbm.at[s, g], dst_i.at[slot], dsem.at[slot])

        def wait_src(g, slot):
            pltpu.make_async_copy(src_hbm.at[s, g], src_i.at[slot],
                                  isem.at[slot]).wait()

        def wait_dst(g, slot):
            pltpu.make_async_copy(dst_hbm.at[s, g], dst_i.at[slot],
                                  dsem.at[slot]).wait()

        def wait_scatter(slot):
            pltpu.make_async_copy(rows_v.at[slot], agg_sp.at[dst_i.at[0]],
                                  ssem.at[slot]).wait()

        for g in range(NBUF):
            stage_src(g, g)
        for g in range(DBUF):
            stage_dst(g, g)
        wait_src(0, 0)
        pltpu.async_copy(h_sp.at[src_i.at[0]], rows_v.at[0], gsem.at[0])

        def step(g, carry):
            b = lax.rem(g, NBUF)
            nb = lax.rem(g + 1, NBUF)
            ds = lax.rem(g, DBUF)

            @pl.when(g + 1 < GROUPS)
            def _():
                wait_src(g + 1, nb)

                @pl.when(g >= 1)
                def _():
                    wait_scatter(nb)

                pltpu.async_copy(h_sp.at[src_i.at[nb]], rows_v.at[nb],
                                 gsem.at[nb])

            pltpu.make_async_copy(h_sp.at[src_i.at[b]], rows_v.at[b],
                                  gsem.at[b]).wait()
            wait_dst(g, ds)
            pltpu.async_copy(rows_v.at[b], agg_sp.at[dst_i.at[ds]],
                             ssem.at[b], add=True)

            @pl.when(g + 2 < GROUPS)
            def _():
                stage_src(g + 2, b)
                stage_dst(g + 2, lax.rem(g + 2, DBUF))

            return carry

        lax.fori_loop(0, GROUPS, step, 0)
        wait_scatter((GROUPS - 1) % NBUF)
        wait_scatter((GROUPS - 2) % NBUF)
        plsc.subcore_barrier()

        pltpu.sync_copy(agg_sp.at[pl.ds(s * BROWS, BROWS)],
                        out_hbm.at[c, pl.ds(s * BROWS, BROWS)])

        @pl.when(s == NS - 1)
        def _():
            pltpu.sync_copy(agg_sp.at[pl.ds(NS * BROWS, BTAIL)],
                            out_hbm.at[c, pl.ds(NS * BROWS, BTAIL)])

    return pl.kernel(
        body,
        out_type=jax.ShapeDtypeStruct((NC, N, HD), jnp.float32),
        mesh=mesh,
        scratch_types=[
            pltpu.VMEM_SHARED((N, HD), jnp.float32),
            pltpu.VMEM_SHARED((N_PAD, HD), jnp.float32),
            pltpu.VMEM((NBUF, CHUNK), jnp.int32),
            pltpu.VMEM((DBUF, CHUNK), jnp.int32),
            pltpu.VMEM((NBUF, CHUNK, HD), jnp.float32),
            pltpu.SemaphoreType.DMA((NBUF,)),
            pltpu.SemaphoreType.DMA((DBUF,)),
            pltpu.SemaphoreType.DMA((NBUF,)),
            pltpu.SemaphoreType.DMA((NBUF,)),
        ],
    )


_sc_seg_sum = _make_sc_seg_sum()


ROWS_BLK = 1000
NB = N // ROWS_BLK


def _tc_layer_body(h_ref, agg_ref, w1_ref, w2_ref, g1_ref, b1_ref,
                   g2_ref, b2_ref, out_ref, hsplit_ref, y_scr, st_scr):
    p = pl.program_id(0)
    i = pl.program_id(1)
    cdims = (((1,), (1,)), ((), ()))

    @pl.when(p == 0)
    def _():
        agg = jnp.concatenate([agg_ref[0], agg_ref[1]], axis=-1)
        x = h_ref[...] + agg
        y = lax.dot_general(x, w1_ref[...], cdims,
                            preferred_element_type=jnp.float32)
        y_scr[pl.ds(i * ROWS_BLK, ROWS_BLK), :] = y
        cs = jnp.sum(y, axis=0, keepdims=True)
        cq = jnp.sum(y * y, axis=0, keepdims=True)

        @pl.when(i == 0)
        def _():
            st_scr[0:1, :] = cs
            st_scr[1:2, :] = cq

        @pl.when(i > 0)
        def _():
            st_scr[0:1, :] += cs
            st_scr[1:2, :] += cq

    @pl.when(p == 1)
    def _():
        m = st_scr[0:1, :] / N
        v = st_scr[1:2, :] / N - m * m
        s1 = g1_ref[...] * lax.rsqrt(v + 1e-5)
        t1 = b1_ref[...] - m * s1
        y = y_scr[pl.ds(i * ROWS_BLK, ROWS_BLK), :]
        u = y * s1 + t1
        u = jnp.where(u >= 0, u, 0.01 * u)
        z = lax.dot_general(u, w2_ref[...], cdims,
                            preferred_element_type=jnp.float32)
        y_scr[pl.ds(i * ROWS_BLK, ROWS_BLK), :] = z
        cs = jnp.sum(z, axis=0, keepdims=True)
        cq = jnp.sum(z * z, axis=0, keepdims=True)

        @pl.when(i == 0)
        def _():
            st_scr[2:3, :] = cs
            st_scr[3:4, :] = cq

        @pl.when(i > 0)
        def _():
            st_scr[2:3, :] += cs
            st_scr[3:4, :] += cq

    @pl.when(p == 2)
    def _():
        m = st_scr[2:3, :] / N
        v = st_scr[3:4, :] / N - m * m
        s2 = g2_ref[...] * lax.rsqrt(v + 1e-5)
        t2 = b2_ref[...] - m * s2
        z = y_scr[pl.ds(i * ROWS_BLK, ROWS_BLK), :]
        o = z * s2 + t2
        o = jnp.where(o >= 0, o, 0.01 * o)
        out_ref[...] = o
        hsplit_ref[0] = o[:, :HD]
        hsplit_ref[1] = o[:, HD:]


def _tc_layer(h, agg2, w1, w2, g1, b1, g2, b2):
    vec = lambda: pl.BlockSpec((1, D), lambda p, i: (0, 0))
    return pl.pallas_call(
        _tc_layer_body,
        grid=(3, NB),
        in_specs=[
            pl.BlockSpec((ROWS_BLK, D),
                         lambda p, i: (jnp.where(p == 0, i, 0), 0)),
            pl.BlockSpec((NC, ROWS_BLK, HD),
                         lambda p, i: (0, jnp.where(p == 0, i, 0), 0)),
            pl.BlockSpec((D, D), lambda p, i: (0, 0)),
            pl.BlockSpec((D, D), lambda p, i: (0, 0)),
            vec(), vec(), vec(), vec(),
        ],
        out_specs=[
            pl.BlockSpec((ROWS_BLK, D),
                         lambda p, i: (jnp.where(p == 2, i, 0), 0)),
            pl.BlockSpec((NC, ROWS_BLK, HD),
                         lambda p, i: (0, jnp.where(p == 2, i, 0), 0)),
        ],
        out_shape=[
            jax.ShapeDtypeStruct((N, D), jnp.float32),
            jax.ShapeDtypeStruct((NC, N, HD), jnp.float32),
        ],
        scratch_shapes=[
            pltpu.VMEM((N, D), jnp.float32),
            pltpu.VMEM((8, D), jnp.float32),
        ],
    )(h, agg2, w1, w2, g1, b1, g2, b2)


def kernel(h, edge_index, W1, W2, mlp_gamma, mlp_beta, out_gamma, out_beta):
    src = edge_index[0]
    dst = edge_index[1]
    pad = NS * EPT_PAD - E
    src_p = jnp.concatenate([src, jnp.zeros((pad,), jnp.int32)])
    dst_p = jnp.concatenate([dst, jnp.full((pad,), N, jnp.int32)])
    src_r = src_p.reshape(NS, GROUPS, CHUNK)
    dst_r = dst_p.reshape(NS, GROUPS, CHUNK)
    zeros_hbm = jnp.zeros((ZROWS, HD), jnp.float32)

    g1 = mlp_gamma.reshape(L, 1, D)
    b1 = mlp_beta.reshape(L, 1, D)
    g2 = out_gamma.reshape(L, 1, D)
    b2 = out_beta.reshape(L, 1, D)

    hh = jnp.stack([h[:, :HD], h[:, HD:]])
    hs = [h]
    for i in range(L):
        agg2 = _sc_seg_sum(hh, src_r, dst_r, zeros_hbm)
        h, hh = _tc_layer(h, agg2, W1[i], W2[i], g1[i], b1[i], g2[i], b2[i])
        hs.append(h)
    return jnp.concatenate(hs, axis=-1)


# speedup vs baseline: 8.1297x; 1.0390x over previous
import jax
import jax.numpy as jnp
from jax import lax
from jax.experimental import pallas as pl
from jax.experimental.pallas import tpu as pltpu
from jax.experimental.pallas import tpu_sc as plsc

N = 10000
E = 320000
D = 128
HD = D // 2
L = 4

NC = 2
NS = 16
CHUNK = 128
NBUF = 2
DBUF = 3
EPT = E // NS
GROUPS = -(-EPT // CHUNK)
EPT_PAD = GROUPS * CHUNK
N_PAD = 10112
ZROWS = N_PAD // NS
BROWS = 624
BTAIL = N - NS * BROWS


def _make_sc_seg_sum():
    mesh = plsc.VectorSubcoreMesh(core_axis_name="c", subcore_axis_name="s",
                                  num_cores=NC)

    def body(hh_hbm, src_hbm, dst_hbm, zeros_hbm, out_hbm,
             h_sp, agg_sp, src_i, dst_i, rows_v, isem, dsem, gsem, ssem):
        c = lax.axis_index("c")
        s = lax.axis_index("s")

        pltpu.sync_copy(zeros_hbm, agg_sp.at[pl.ds(s * ZROWS, ZROWS)])
        pltpu.sync_copy(hh_hbm.at[c, pl.ds(s * BROWS, BROWS)],
                        h_sp.at[pl.ds(s * BROWS, BROWS)])

        @pl.when(s == NS - 1)
        def _():
            pltpu.sync_copy(hh_hbm.at[c, pl.ds(NS * BROWS, BTAIL)],
                            h_sp.at[pl.ds(NS * BROWS, BTAIL)])

        plsc.subcore_barrier()

        def stage_src(g, slot):
            pltpu.async_copy(src_hbm.at[s, g], src_i.at[slot], isem.at[slot])

        def stage_dst(g, slot):
            pltpu.async_copy(dst_hbm.at[s, g], dst_i.at[slot], dsem.at[slot])

        def wait_src(g, slot):
            pltpu.make_async_copy(src_hbm.at[s, g], src_i.at[slot],
                                  isem.at[slot]).wait()

        def wait_dst(g, slot):
            pltpu.make_async_copy(dst_hbm.at[s, g], dst_i.at[slot],
                                  dsem.at[slot]).wait()

        def wait_scatter(slot):
            pltpu.make_async_copy(rows_v.at[slot], agg_sp.at[dst_i.at[0]],
                                  ssem.at[slot]).wait()

        for g in range(NBUF):
            stage_src(g, g)
        for g in range(DBUF):
            stage_dst(g, g)
        wait_src(0, 0)
        pltpu.async_copy(h_sp.at[src_i.at[0]], rows_v.at[0], gsem.at[0])

        def step(g, carry):
            b = lax.rem(g, NBUF)
            nb = lax.rem(g + 1, NBUF)
            ds = lax.rem(g, DBUF)

            @pl.when(g + 1 < GROUPS)
            def _():
                wait_src(g + 1, nb)

                @pl.when(g >= 1)
                def _():
                    wait_scatter(nb)

                pltpu.async_copy(h_sp.at[src_i.at[nb]], rows_v.at[nb],
                                 gsem.at[nb])

            pltpu.make_async_copy(h_sp.at[src_i.at[b]], rows_v.at[b],
                                  gsem.at[b]).wait()
            wait_dst(g, ds)
            pltpu.async_copy(rows_v.at[b], agg_sp.at[dst_i.at[ds]],
                             ssem.at[b], add=True)

            @pl.when(g + 2 < GROUPS)
            def _():
                stage_src(g + 2, b)
                stage_dst(g + 2, lax.rem(g + 2, DBUF))

            return carry

        lax.fori_loop(0, GROUPS, step, 0)
        wait_scatter((GROUPS - 1) % NBUF)
        wait_scatter((GROUPS - 2) % NBUF)
        plsc.subcore_barrier()

        pltpu.sync_copy(agg_sp.at[pl.ds(s * BROWS, BROWS)],
                        out_hbm.at[c, pl.ds(s * BROWS, BROWS)])

        @pl.when(s == NS - 1)
        def _():
            pltpu.sync_copy(agg_sp.at[pl.ds(NS * BROWS, BTAIL)],
                            out_hbm.at[c, pl.ds(NS * BROWS, BTAIL)])

    return pl.kernel(
        body,
        out_type=jax.ShapeDtypeStruct((NC, N, HD), jnp.float32),
        mesh=mesh,
        scratch_types=[
            pltpu.VMEM_SHARED((N, HD), jnp.float32),
            pltpu.VMEM_SHARED((N_PAD, HD), jnp.float32),
            pltpu.VMEM((NBUF, CHUNK), jnp.int32),
            pltpu.VMEM((DBUF, CHUNK), jnp.int32),
            pltpu.VMEM((NBUF, CHUNK, HD), jnp.float32),
            pltpu.SemaphoreType.DMA((NBUF,)),
            pltpu.SemaphoreType.DMA((DBUF,)),
            pltpu.SemaphoreType.DMA((NBUF,)),
            pltpu.SemaphoreType.DMA((NBUF,)),
        ],
    )


_sc_seg_sum = _make_sc_seg_sum()


ROWS_BLK = 2000
NB = N // ROWS_BLK


def _tc_layer_body(h_ref, agg_ref, w1_ref, w2_ref, g1_ref, b1_ref,
                   g2_ref, b2_ref, out_ref, hsplit_ref, y_scr, st_scr):
    p = pl.program_id(0)
    i = pl.program_id(1)
    cdims = (((1,), (1,)), ((), ()))

    @pl.when(p == 0)
    def _():
        agg = jnp.concatenate([agg_ref[0], agg_ref[1]], axis=-1)
        x = h_ref[...] + agg
        y = lax.dot_general(x, w1_ref[...], cdims,
                            preferred_element_type=jnp.float32)
        y_scr[pl.ds(i * ROWS_BLK, ROWS_BLK), :] = y
        cs = jnp.sum(y, axis=0, keepdims=True)
        cq = jnp.sum(y * y, axis=0, keepdims=True)

        @pl.when(i == 0)
        def _():
            st_scr[0:1, :] = cs
            st_scr[1:2, :] = cq

        @pl.when(i > 0)
        def _():
            st_scr[0:1, :] += cs
            st_scr[1:2, :] += cq

    @pl.when(p == 1)
    def _():
        m = st_scr[0:1, :] / N
        v = st_scr[1:2, :] / N - m * m
        s1 = g1_ref[...] * lax.rsqrt(v + 1e-5)
        t1 = b1_ref[...] - m * s1
        y = y_scr[pl.ds(i * ROWS_BLK, ROWS_BLK), :]
        u = y * s1 + t1
        u = jnp.where(u >= 0, u, 0.01 * u)
        z = lax.dot_general(u, w2_ref[...], cdims,
                            preferred_element_type=jnp.float32)
        y_scr[pl.ds(i * ROWS_BLK, ROWS_BLK), :] = z
        cs = jnp.sum(z, axis=0, keepdims=True)
        cq = jnp.sum(z * z, axis=0, keepdims=True)

        @pl.when(i == 0)
        def _():
            st_scr[2:3, :] = cs
            st_scr[3:4, :] = cq

        @pl.when(i > 0)
        def _():
            st_scr[2:3, :] += cs
            st_scr[3:4, :] += cq

    @pl.when(p == 2)
    def _():
        m = st_scr[2:3, :] / N
        v = st_scr[3:4, :] / N - m * m
        s2 = g2_ref[...] * lax.rsqrt(v + 1e-5)
        t2 = b2_ref[...] - m * s2
        z = y_scr[pl.ds(i * ROWS_BLK, ROWS_BLK), :]
        o = z * s2 + t2
        o = jnp.where(o >= 0, o, 0.01 * o)
        out_ref[...] = o
        hsplit_ref[0] = o[:, :HD]
        hsplit_ref[1] = o[:, HD:]


def _tc_layer(h, agg2, w1, w2, g1, b1, g2, b2):
    vec = lambda: pl.BlockSpec((1, D), lambda p, i: (0, 0))
    return pl.pallas_call(
        _tc_layer_body,
        grid=(3, NB),
        in_specs=[
            pl.BlockSpec((ROWS_BLK, D),
                         lambda p, i: (jnp.where(p == 0, i, 0), 0)),
            pl.BlockSpec((NC, ROWS_BLK, HD),
                         lambda p, i: (0, jnp.where(p == 0, i, 0), 0)),
            pl.BlockSpec((D, D), lambda p, i: (0, 0)),
            pl.BlockSpec((D, D), lambda p, i: (0, 0)),
            vec(), vec(), vec(), vec(),
        ],
        out_specs=[
            pl.BlockSpec((ROWS_BLK, D),
                         lambda p, i: (jnp.where(p == 2, i, 0), 0)),
            pl.BlockSpec((NC, ROWS_BLK, HD),
                         lambda p, i: (0, jnp.where(p == 2, i, 0), 0)),
        ],
        out_shape=[
            jax.ShapeDtypeStruct((N, D), jnp.float32),
            jax.ShapeDtypeStruct((NC, N, HD), jnp.float32),
        ],
        scratch_shapes=[
            pltpu.VMEM((N, D), jnp.float32),
            pltpu.VMEM((8, D), jnp.float32),
        ],
    )(h, agg2, w1, w2, g1, b1, g2, b2)


def kernel(h, edge_index, W1, W2, mlp_gamma, mlp_beta, out_gamma, out_beta):
    src = edge_index[0]
    dst = edge_index[1]
    pad = NS * EPT_PAD - E
    src_p = jnp.concatenate([src, jnp.zeros((pad,), jnp.int32)])
    dst_p = jnp.concatenate([dst, jnp.full((pad,), N, jnp.int32)])
    src_r = src_p.reshape(NS, GROUPS, CHUNK)
    dst_r = dst_p.reshape(NS, GROUPS, CHUNK)
    zeros_hbm = jnp.zeros((ZROWS, HD), jnp.float32)

    g1 = mlp_gamma.reshape(L, 1, D)
    b1 = mlp_beta.reshape(L, 1, D)
    g2 = out_gamma.reshape(L, 1, D)
    b2 = out_beta.reshape(L, 1, D)

    hh = jnp.stack([h[:, :HD], h[:, HD:]])
    hs = [h]
    for i in range(L):
        agg2 = _sc_seg_sum(hh, src_r, dst_r, zeros_hbm)
        h, hh = _tc_layer(h, agg2, W1[i], W2[i], g1[i], b1[i], g2[i], b2[i])
        hs.append(h)
    return jnp.concatenate(hs, axis=-1)
